# baseline (speedup 1.0000x reference)
import functools

import jax
import jax.numpy as jnp
from jax.experimental import pallas as pl
from jax.experimental.pallas import tpu as pltpu
from jax.experimental.pallas import tpu_sc as plsc


def _prep_kernel(x_ref, u_ref, w_ref, b_ref, emb_ref, midx_ref):
    n_b, m = u_ref.shape
    emb = jnp.dot(x_ref[...], w_ref[...], preferred_element_type=jnp.float32)
    emb_ref[...] = emb + b_ref[...]
    u = u_ref[...]
    col = jax.lax.broadcasted_iota(jnp.int32, (n_b, m), 1)
    umin = jnp.min(u, axis=1, keepdims=True)
    midx_ref[...] = jnp.min(jnp.where(u == umin, col, m), axis=1, keepdims=True)


def _make_sc_copy(B, M, D):
    info = plsc.get_sparse_core_info()
    nc, ns = info.num_cores, info.num_subcores
    nw = nc * ns
    bpw = B // nw
    rows_c = 256
    cpb = M // rows_c
    nch = bpw * cpb
    nbuf = 4

    mesh = plsc.VectorSubcoreMesh(core_axis_name="c", subcore_axis_name="s")

    @functools.partial(
        pl.kernel,
        mesh=mesh,
        out_type=jax.ShapeDtypeStruct((B, M, D), jnp.float32),
        scratch_types=[
            pltpu.VMEM((rows_c, D), jnp.float32),
            pltpu.VMEM((rows_c, D), jnp.float32),
            pltpu.VMEM((rows_c, D), jnp.float32),
            pltpu.VMEM((rows_c, D), jnp.float32),
            pltpu.SemaphoreType.DMA((nbuf,)),
            pltpu.SemaphoreType.DMA((nbuf,)),
        ],
    )
    def sc_body(tv_hbm, out_hbm, buf0, buf1, buf2, buf3, gsem, ssem):
        wid = jax.lax.axis_index("s") * nc + jax.lax.axis_index("c")
        base_b = wid * bpw
        bufs = (buf0, buf1, buf2, buf3)

        def gather(g):
            return pltpu.make_async_copy(
                tv_hbm.at[base_b + g // cpb, pl.ds((g % cpb) * rows_c, rows_c), :],
                bufs[g % nbuf], gsem.at[g % nbuf])

        def scatter(g):
            return pltpu.make_async_copy(
                bufs[g % nbuf],
                out_hbm.at[base_b + g // cpb, pl.ds((g % cpb) * rows_c, rows_c), :],
                ssem.at[g % nbuf])

        for g in range(nch):
            if g >= nbuf:
                scatter(g - nbuf).wait()
            gather(g).start()
            if g >= 2:
                gather(g - 2).wait()
                scatter(g - 2).start()
        for g in range(nch - 2, nch):
            gather(g).wait()
            scatter(g).start()
        for g in range(nch - nbuf, nch):
            scatter(g).wait()

    return sc_body


_W = 32


def _patch_kernel(midx_ref, emb_ref, base_ref, out_ref, sems):
    del base_ref
    n_b = emb_ref.shape[0]

    def patch(b, slot):
        s = midx_ref[b, 0]
        return pltpu.make_async_copy(
            emb_ref.at[pl.ds(b, 1), :, :],
            out_ref.at[pl.ds(b, 1), pl.ds(s, 1), :],
            sems.at[slot],
        )

    def issue(i, _):
        for k in range(8):
            patch(i * 8 + k, 0).start()
        return ()

    def drain(i, _):
        for k in range(8):
            patch(i * 8 + k, 0).wait()
        return ()

    jax.lax.fori_loop(0, n_b // 8, issue, ())
    jax.lax.fori_loop(0, n_b // 8, drain, ())


def kernel(x, token_values, token_usages, W_embed, b_embed):
    B, M, D = token_values.shape
    emb, midx = pl.pallas_call(
        _prep_kernel,
        grid=(1,),
        in_specs=[
            pl.BlockSpec((B, D), lambda i: (0, 0)),
            pl.BlockSpec((B, M), lambda i: (0, 0)),
            pl.BlockSpec((D, D), lambda i: (0, 0)),
            pl.BlockSpec((1, D), lambda i: (0, 0)),
        ],
        out_specs=[
            pl.BlockSpec((B, D), lambda i: (0, 0)),
            pl.BlockSpec((B, 1), lambda i: (0, 0)),
        ],
        out_shape=[
            jax.ShapeDtypeStruct((B, D), jnp.float32),
            jax.ShapeDtypeStruct((B, 1), jnp.int32),
        ],
    )(x, token_usages, W_embed, b_embed.reshape(1, D))
    copied = _make_sc_copy(B, M, D)(token_values)
    out = pl.pallas_call(
        _patch_kernel,
        grid=(1,),
        in_specs=[
            pl.BlockSpec(memory_space=pltpu.MemorySpace.SMEM),
            pl.BlockSpec((B, 1, D), lambda i: (0, 0, 0)),
            pl.BlockSpec(memory_space=pl.ANY),
        ],
        out_specs=pl.BlockSpec(memory_space=pl.ANY),
        out_shape=jax.ShapeDtypeStruct((B, M, D), jnp.float32),
        input_output_aliases={2: 0},
        scratch_shapes=[pltpu.SemaphoreType.DMA((_W,))],
    )(midx, emb.reshape(B, 1, D), copied)
    return out


# speedup vs baseline: 1.0006x; 1.0006x over previous
import functools

import jax
import jax.numpy as jnp
from jax.experimental import pallas as pl
from jax.experimental.pallas import tpu as pltpu
from jax.experimental.pallas import tpu_sc as plsc


def _prep_kernel(x_ref, u_ref, w_ref, b_ref, emb_ref, midx_ref):
    n_b, m = u_ref.shape
    emb = jnp.dot(x_ref[...], w_ref[...], preferred_element_type=jnp.float32)
    emb_ref[...] = emb + b_ref[...]
    u = u_ref[...]
    col = jax.lax.broadcasted_iota(jnp.int32, (n_b, m), 1)
    umin = jnp.min(u, axis=1, keepdims=True)
    midx_ref[...] = jnp.min(jnp.where(u == umin, col, m), axis=1, keepdims=True)


def _make_sc_copy(B, M, D):
    info = plsc.get_sparse_core_info()
    nc, ns = info.num_cores, info.num_subcores
    nw = nc * ns
    bpw = B // nw
    rows_c = 256
    cpb = M // rows_c
    nch = bpw * cpb
    nbuf = 4

    mesh = plsc.VectorSubcoreMesh(core_axis_name="c", subcore_axis_name="s")

    @functools.partial(
        pl.kernel,
        mesh=mesh,
        out_type=jax.ShapeDtypeStruct((B, M, D), jnp.float32),
        scratch_types=[
            pltpu.VMEM((rows_c, D), jnp.float32),
            pltpu.VMEM((rows_c, D), jnp.float32),
            pltpu.VMEM((rows_c, D), jnp.float32),
            pltpu.VMEM((rows_c, D), jnp.float32),
            pltpu.SemaphoreType.DMA((nbuf,)),
            pltpu.SemaphoreType.DMA((nbuf,)),
        ],
    )
    def sc_body(tv_hbm, out_hbm, buf0, buf1, buf2, buf3, gsem, ssem):
        wid = jax.lax.axis_index("s") * nc + jax.lax.axis_index("c")
        base_b = wid * bpw
        bufs = (buf0, buf1, buf2, buf3)

        def gather(g):
            return pltpu.make_async_copy(
                tv_hbm.at[base_b + g // cpb, pl.ds((g % cpb) * rows_c, rows_c), :],
                bufs[g % nbuf], gsem.at[g % nbuf])

        def scatter(g):
            return pltpu.make_async_copy(
                bufs[g % nbuf],
                out_hbm.at[base_b + g // cpb, pl.ds((g % cpb) * rows_c, rows_c), :],
                ssem.at[g % nbuf])

        for g in range(nch):
            if g >= nbuf:
                scatter(g - nbuf).wait()
            gather(g).start()
            if g >= 2:
                gather(g - 2).wait()
                scatter(g - 2).start()
        for g in range(nch - 2, nch):
            gather(g).wait()
            scatter(g).start()
        for g in range(nch - nbuf, nch):
            scatter(g).wait()

    return sc_body


def _patch_kernel(midx_ref, emb_ref, base_ref, out_ref, sem):
    del base_ref
    n_b = emb_ref.shape[0]

    def patch(b):
        s = midx_ref[b, 0]
        return pltpu.make_async_copy(
            emb_ref.at[pl.ds(b, 1), :, :],
            out_ref.at[pl.ds(b, 1), pl.ds(s, 1), :],
            sem,
        )

    def issue(i, _):
        for k in range(8):
            patch(i * 8 + k).start()
        return ()

    def drain(i, _):
        for k in range(8):
            patch(i * 8 + k).wait()
        return ()

    jax.lax.fori_loop(0, n_b // 8, issue, ())
    jax.lax.fori_loop(0, n_b // 8, drain, ())


def kernel(x, token_values, token_usages, W_embed, b_embed):
    B, M, D = token_values.shape
    emb, midx = pl.pallas_call(
        _prep_kernel,
        grid=(1,),
        in_specs=[
            pl.BlockSpec((B, D), lambda i: (0, 0)),
            pl.BlockSpec((B, M), lambda i: (0, 0)),
            pl.BlockSpec((D, D), lambda i: (0, 0)),
            pl.BlockSpec((1, D), lambda i: (0, 0)),
        ],
        out_specs=[
            pl.BlockSpec((B, D), lambda i: (0, 0)),
            pl.BlockSpec((B, 1), lambda i: (0, 0)),
        ],
        out_shape=[
            jax.ShapeDtypeStruct((B, D), jnp.float32),
            jax.ShapeDtypeStruct((B, 1), jnp.int32),
        ],
    )(x, token_usages, W_embed, b_embed.reshape(1, D))
    copied = _make_sc_copy(B, M, D)(token_values)
    out = pl.pallas_call(
        _patch_kernel,
        grid=(1,),
        in_specs=[
            pl.BlockSpec(memory_space=pltpu.MemorySpace.SMEM),
            pl.BlockSpec((B, 1, D), lambda i: (0, 0, 0)),
            pl.BlockSpec(memory_space=pl.ANY),
        ],
        out_specs=pl.BlockSpec(memory_space=pl.ANY),
        out_shape=jax.ShapeDtypeStruct((B, M, D), jnp.float32),
        input_output_aliases={2: 0},
        scratch_shapes=[pltpu.SemaphoreType.DMA],
    )(midx, emb.reshape(B, 1, D), copied)
    return out
